# Initial kernel scaffold; baseline (speedup 1.0000x reference)
import jax
import jax.numpy as jnp
from jax import lax
from jax.experimental import pallas as pl
from jax.experimental.pallas import tpu as pltpu
from jax.experimental.pallas import tpu_sc as plsc

NC = 2
NS = 16
CH = 128
SB = 8


def _leaky(v, s):
    return jnp.where(v >= 0, v, s * v)


def _nan0(v):
    return jnp.where(jnp.isnan(v), 0.0, v)


def _edge_mlp_body(ea_ref, w1_ref, b1_ref, w2_ref, b2_ref, out_ref):
    a = _nan0(ea_ref[...])
    h = _leaky(jnp.dot(a, w1_ref[...], preferred_element_type=jnp.float32)
               + b1_ref[...], 0.2)
    o = _leaky(jnp.dot(h, w2_ref[...], preferred_element_type=jnp.float32)
               + b2_ref[...], 0.005)
    out_ref[...] = o


def _edge_mlp(ea, w1, b1, w2, b2):
    E = ea.shape[0]
    BE = 16000
    grid = E // BE
    full = lambda shp: pl.BlockSpec(shp, lambda i: (0,) * len(shp))
    return pl.pallas_call(
        _edge_mlp_body,
        grid=(grid,),
        in_specs=[
            pl.BlockSpec((BE, 4), lambda i: (i, 0)),
            full((4, 16)), full((1, 16)), full((16, 1)), full((1, 1)),
        ],
        out_specs=pl.BlockSpec((BE, 1), lambda i: (i, 0)),
        out_shape=jax.ShapeDtypeStruct((E, 1), jnp.float32),
    )(ea, w1, b1, w2, b2)


def _pre_body(x_ref, degp_ref, w1_ref, b1_ref, w2_ref, b2_ref, w0_ref,
              y_ref, dinv_ref):
    xb = _nan0(x_ref[...])
    n1 = _leaky(jnp.dot(xb, w1_ref[...], preferred_element_type=jnp.float32)
                + b1_ref[...], 0.2)
    n2 = _leaky(jnp.dot(n1, w2_ref[...], preferred_element_type=jnp.float32)
                + b2_ref[...], 0.2)
    deg = 1.0 + degp_ref[0] + degp_ref[1]
    dv = jnp.where(deg == 0.0, 0.0, lax.rsqrt(deg))
    y_ref[...] = dv * jnp.dot(n2, w0_ref[...],
                              preferred_element_type=jnp.float32)
    dinv_ref[...] = dv


def _pre(x, degp, w1, b1, w2, b2, w0):
    N = x.shape[0]
    B = 2000
    grid = N // B
    full = lambda shp: pl.BlockSpec(shp, lambda i: (0,) * len(shp))
    return pl.pallas_call(
        _pre_body,
        grid=(grid,),
        in_specs=[
            pl.BlockSpec((B, 7), lambda i: (i, 0)),
            pl.BlockSpec((2, B, 1), lambda i: (0, i, 0)),
            full((7, 64)), full((1, 64)), full((64, 64)), full((1, 64)),
            full((64, 64)),
        ],
        out_specs=[
            pl.BlockSpec((B, 64), lambda i: (i, 0)),
            pl.BlockSpec((B, 1), lambda i: (i, 0)),
        ],
        out_shape=[
            jax.ShapeDtypeStruct((N, 64), jnp.float32),
            jax.ShapeDtypeStruct((N, 1), jnp.float32),
        ],
    )(x, degp, w1, b1, w2, b2, w0)


def _layer_body(agg_ref, y_ref, dinv_ref, b_ref, w_ref, out_ref):
    a = jnp.concatenate([agg_ref[0], agg_ref[1]], axis=1)
    dv = dinv_ref[...]
    node = _leaky(dv * (a + y_ref[...]) + b_ref[...], 0.2)
    out_ref[...] = dv * jnp.dot(node, w_ref[...],
                                preferred_element_type=jnp.float32)


def _layer(agg, y, dinv, b, w):
    N = y.shape[0]
    B = 2000
    grid = N // B
    full = lambda shp: pl.BlockSpec(shp, lambda i: (0,) * len(shp))
    return pl.pallas_call(
        _layer_body,
        grid=(grid,),
        in_specs=[
            pl.BlockSpec((2, B, 32), lambda i: (0, i, 0)),
            pl.BlockSpec((B, 64), lambda i: (i, 0)),
            pl.BlockSpec((B, 1), lambda i: (i, 0)),
            full((1, 64)), full((64, 64)),
        ],
        out_specs=pl.BlockSpec((B, 64), lambda i: (i, 0)),
        out_shape=jax.ShapeDtypeStruct((N, 64), jnp.float32),
    )(agg, y, dinv, b, w)


def _final_body(agg_ref, y_ref, dinv_ref, b_ref, wp1_ref, bp1_ref,
                wp2_ref, bp2_ref, wr_ref, br_ref, out_ref):
    a = jnp.concatenate([agg_ref[0], agg_ref[1]], axis=1)
    node = _leaky(dinv_ref[...] * (a + y_ref[...]) + b_ref[...], 0.2)
    h = _leaky(jnp.dot(node, wp1_ref[...], preferred_element_type=jnp.float32)
               + bp1_ref[...], 0.2)
    h = _leaky(jnp.dot(h, wp2_ref[...], preferred_element_type=jnp.float32)
               + bp2_ref[...], 0.2)
    out_ref[...] = jnp.dot(h, wr_ref[...],
                           preferred_element_type=jnp.float32) + br_ref[...]


def _final(agg, y, dinv, b, wp1, bp1, wp2, bp2, wr, br):
    N = y.shape[0]
    B = 2000
    grid = N // B
    full = lambda shp: pl.BlockSpec(shp, lambda i: (0,) * len(shp))
    return pl.pallas_call(
        _final_body,
        grid=(grid,),
        in_specs=[
            pl.BlockSpec((2, B, 32), lambda i: (0, i, 0)),
            pl.BlockSpec((B, 64), lambda i: (i, 0)),
            pl.BlockSpec((B, 1), lambda i: (i, 0)),
            full((1, 64)), full((64, 64)), full((1, 64)),
            full((64, 64)), full((1, 64)), full((64, 4)), full((1, 4)),
        ],
        out_specs=pl.BlockSpec((B, 4), lambda i: (i, 0)),
        out_shape=jax.ShapeDtypeStruct((N, 4), jnp.float32),
    )(agg, y, dinv, b, wp1, bp1, wp2, bp2, wr, br)


def _sc_deg(col2d, e2d, zd, n_pad, interpret=False):
    nchunk = col2d.shape[0]
    per_worker = nchunk // (NC * NS)
    dsb = 4
    iters = per_worker // dsb
    rt = n_pad // NS

    def body(col_hbm, e_hbm, zd_hbm, out_hbm, cbuf, ebuf, deg_sh, sem):
        c = lax.axis_index("c")
        s = lax.axis_index("s")
        lo = s * rt
        pltpu.sync_copy(zd_hbm.at[pl.ds(lo, rt)], deg_sh.at[pl.ds(lo, rt)])
        plsc.subcore_barrier()

        wid = s * NC + c
        base0 = wid * per_worker

        def it_body(it, carry):
            base = base0 + it * dsb
            pltpu.sync_copy(col_hbm.at[pl.ds(base, dsb)], cbuf)
            pltpu.sync_copy(e_hbm.at[pl.ds(base, dsb)], ebuf)
            for j in range(dsb):
                pltpu.sync_copy(ebuf.at[j], deg_sh.at[cbuf.at[j]], add=True)
            return carry

        lax.fori_loop(0, iters, it_body, 0)
        plsc.subcore_barrier()
        pltpu.sync_copy(deg_sh.at[pl.ds(lo, rt)],
                        out_hbm.at[c, pl.ds(lo, rt)])

    mesh = plsc.VectorSubcoreMesh(core_axis_name="c", subcore_axis_name="s",
                                  num_cores=NC, num_subcores=NS)
    return pl.kernel(
        body,
        out_type=jax.ShapeDtypeStruct((NC, n_pad), jnp.float32),
        mesh=mesh,
        scratch_types=[
            pltpu.VMEM((dsb, CH), jnp.int32),
            pltpu.VMEM((dsb, CH), jnp.float32),
            pltpu.VMEM_SHARED((n_pad,), jnp.float32),
            pltpu.SemaphoreType.DMA,
        ],
        interpret=interpret,
    )(col2d, e2d, zd)


def _sc_agg(yflat, row2d, col2d, e2d, za, n_pad, interpret=False):
    nchunk = row2d.shape[0]
    per_tile = nchunk // NS
    iters = per_tile // SB
    rt = n_pad // NS

    def body(y_hbm, row_hbm, col_hbm, e_hbm, za_hbm, out_hbm,
             rbuf, cbuf, ebuf, ibuf, rows, agg_sh, sem):
        c = lax.axis_index("c")
        s = lax.axis_index("s")
        lo = s * rt
        pltpu.sync_copy(za_hbm.at[pl.ds(lo, rt)], agg_sh.at[pl.ds(lo, rt)])
        plsc.subcore_barrier()

        base0 = s * per_tile

        def it_body(it, carry):
            base = base0 + it * SB
            pltpu.sync_copy(row_hbm.at[pl.ds(base, SB)], rbuf)
            pltpu.sync_copy(col_hbm.at[pl.ds(base, SB)], cbuf)
            pltpu.sync_copy(e_hbm.at[pl.ds(base, SB)], ebuf)
            for j in range(SB):
                for l in range(CH // 16):
                    v = rbuf[j, pl.ds(l * 16, 16)]
                    ibuf[j, pl.ds(l * 16, 16)] = v + v + c
                pltpu.async_copy(y_hbm.at[ibuf.at[j]], rows, sem).wait()

                def scale_body(i, carry2):
                    w = ebuf[j, i]
                    rows[i, pl.ds(0, 16)] = rows[i, pl.ds(0, 16)] * w
                    rows[i, pl.ds(16, 16)] = rows[i, pl.ds(16, 16)] * w
                    return carry2

                lax.fori_loop(0, CH, scale_body, 0, unroll=8)
                pltpu.sync_copy(rows, agg_sh.at[cbuf.at[j]], add=True)
            return carry

        lax.fori_loop(0, iters, it_body, 0)
        plsc.subcore_barrier()
        pltpu.sync_copy(agg_sh.at[pl.ds(lo, rt)],
                        out_hbm.at[c, pl.ds(lo, rt)])

    mesh = plsc.VectorSubcoreMesh(core_axis_name="c", subcore_axis_name="s",
                                  num_cores=NC, num_subcores=NS)
    return pl.kernel(
        body,
        out_type=jax.ShapeDtypeStruct((NC, n_pad, 32), jnp.float32),
        mesh=mesh,
        scratch_types=[
            pltpu.VMEM((SB, CH), jnp.int32),
            pltpu.VMEM((SB, CH), jnp.int32),
            pltpu.VMEM((SB, CH), jnp.float32),
            pltpu.VMEM((SB, CH), jnp.int32),
            pltpu.VMEM((CH, 32), jnp.float32),
            pltpu.VMEM_SHARED((n_pad, 32), jnp.float32),
            pltpu.SemaphoreType.DMA,
        ],
        interpret=interpret,
    )(yflat, row2d, col2d, e2d, za)


def _gcn_forward(x, edge_index, edge_attr, W1n, b1n, W2n, b2n, W1e, b1e,
                 W2e, b2e, gcn_W, gcn_b, Wp1, bp1, Wp2, bp2, Wr, br,
                 interpret=False):
    N = x.shape[0]
    E = edge_attr.shape[0]

    rt = ((-(-N // NS)) + 7) // 8 * 8
    n_pad = rt * NS

    blk = NS * SB * CH
    e_pad = -(-E // blk) * blk
    nchunk = e_pad // CH
    pad = e_pad - E

    row = edge_index[0].astype(jnp.int32)
    col = edge_index[1].astype(jnp.int32)
    row2d = jnp.concatenate([row, jnp.zeros((pad,), jnp.int32)]).reshape(
        nchunk, CH)
    col2d = jnp.concatenate([col, jnp.zeros((pad,), jnp.int32)]).reshape(
        nchunk, CH)

    e = _edge_mlp(edge_attr, W1e, b1e.reshape(1, 16), W2e,
                  b2e.reshape(1, 1))
    e2d = jnp.concatenate([e.reshape(-1),
                           jnp.zeros((pad,), jnp.float32)]).reshape(nchunk, CH)

    zd = jnp.zeros((n_pad,), jnp.float32)
    za = jnp.zeros((n_pad, 32), jnp.float32)

    degp = _sc_deg(col2d, e2d, zd, n_pad, interpret=interpret)
    degp = degp[:, :N].reshape(NC, N, 1)

    y, dinv = _pre(x, degp, W1n, b1n.reshape(1, 64), W2n,
                   b2n.reshape(1, 64), gcn_W[0])

    out = None
    for i in range(8):
        yflat = y.reshape(2 * N, 32)
        agg = _sc_agg(yflat, row2d, col2d, e2d, za, n_pad,
                      interpret=interpret)
        agg = agg[:, :N]
        b = gcn_b[i].reshape(1, 64)
        if i < 7:
            y = _layer(agg, y, dinv, b, gcn_W[i + 1])
        else:
            out = _final(agg, y, dinv, b, Wp1, bp1.reshape(1, 64),
                         Wp2, bp2.reshape(1, 64), Wr, br.reshape(1, 4))
    return out


def kernel(x, edge_index, edge_attr, W1n, b1n, W2n, b2n, W1e, b1e, W2e,
           b2e, gcn_W, gcn_b, Wp1, bp1, Wp2, bp2, Wr, br):
    return _gcn_forward(x, edge_index, edge_attr, W1n, b1n, W2n, b2n,
                        W1e, b1e, W2e, b2e, gcn_W, gcn_b, Wp1, bp1,
                        Wp2, bp2, Wr, br)


# speedup vs baseline: 12.1363x; 12.1363x over previous
import jax
import jax.numpy as jnp
from jax import lax
from jax.experimental import pallas as pl
from jax.experimental.pallas import tpu as pltpu
from jax.experimental.pallas import tpu_sc as plsc

NC = 2
NS = 16
CH = 256
SB = 1


def _leaky(v, s):
    return jnp.where(v >= 0, v, s * v)


def _nan0(v):
    return jnp.where(jnp.isnan(v), 0.0, v)


def _edge_mlp_body(ea_ref, w1_ref, b1_ref, w2_ref, b2_ref, out_ref):
    a = _nan0(ea_ref[...])
    h = _leaky(jnp.dot(a, w1_ref[...], preferred_element_type=jnp.float32)
               + b1_ref[...], 0.2)
    o = _leaky(jnp.dot(h, w2_ref[...], preferred_element_type=jnp.float32)
               + b2_ref[...], 0.005)
    out_ref[...] = o


def _edge_mlp(ea, w1, b1, w2, b2):
    E = ea.shape[0]
    BE = 16000
    grid = E // BE
    full = lambda shp: pl.BlockSpec(shp, lambda i: (0,) * len(shp))
    return pl.pallas_call(
        _edge_mlp_body,
        grid=(grid,),
        in_specs=[
            pl.BlockSpec((BE, 4), lambda i: (i, 0)),
            full((4, 16)), full((1, 16)), full((16, 1)), full((1, 1)),
        ],
        out_specs=pl.BlockSpec((BE, 1), lambda i: (i, 0)),
        out_shape=jax.ShapeDtypeStruct((E, 1), jnp.float32),
    )(ea, w1, b1, w2, b2)


def _pre_body(x_ref, degp_ref, w1_ref, b1_ref, w2_ref, b2_ref, w0_ref,
              y_ref, dinv_ref):
    xb = _nan0(x_ref[...])
    n1 = _leaky(jnp.dot(xb, w1_ref[...], preferred_element_type=jnp.float32)
                + b1_ref[...], 0.2)
    n2 = _leaky(jnp.dot(n1, w2_ref[...], preferred_element_type=jnp.float32)
                + b2_ref[...], 0.2)
    deg = 1.0 + degp_ref[0] + degp_ref[1]
    dv = jnp.where(deg == 0.0, 0.0, lax.rsqrt(deg))
    y_ref[...] = dv * jnp.dot(n2, w0_ref[...],
                              preferred_element_type=jnp.float32)
    dinv_ref[...] = dv


def _pre(x, degp, w1, b1, w2, b2, w0):
    N = x.shape[0]
    B = 2000
    grid = N // B
    full = lambda shp: pl.BlockSpec(shp, lambda i: (0,) * len(shp))
    return pl.pallas_call(
        _pre_body,
        grid=(grid,),
        in_specs=[
            pl.BlockSpec((B, 7), lambda i: (i, 0)),
            pl.BlockSpec((2, B, 1), lambda i: (0, i, 0)),
            full((7, 64)), full((1, 64)), full((64, 64)), full((1, 64)),
            full((64, 64)),
        ],
        out_specs=[
            pl.BlockSpec((B, 64), lambda i: (i, 0)),
            pl.BlockSpec((B, 1), lambda i: (i, 0)),
        ],
        out_shape=[
            jax.ShapeDtypeStruct((N, 64), jnp.float32),
            jax.ShapeDtypeStruct((N, 1), jnp.float32),
        ],
    )(x, degp, w1, b1, w2, b2, w0)


def _layer_body(agg_ref, y_ref, dinv_ref, b_ref, w_ref, out_ref):
    a = jnp.concatenate([agg_ref[0], agg_ref[1]], axis=1)
    dv = dinv_ref[...]
    node = _leaky(dv * (a + y_ref[...]) + b_ref[...], 0.2)
    out_ref[...] = dv * jnp.dot(node, w_ref[...],
                                preferred_element_type=jnp.float32)


def _layer(agg, y, dinv, b, w):
    N = y.shape[0]
    B = 2000
    grid = N // B
    full = lambda shp: pl.BlockSpec(shp, lambda i: (0,) * len(shp))
    return pl.pallas_call(
        _layer_body,
        grid=(grid,),
        in_specs=[
            pl.BlockSpec((2, B, 32), lambda i: (0, i, 0)),
            pl.BlockSpec((B, 64), lambda i: (i, 0)),
            pl.BlockSpec((B, 1), lambda i: (i, 0)),
            full((1, 64)), full((64, 64)),
        ],
        out_specs=pl.BlockSpec((B, 64), lambda i: (i, 0)),
        out_shape=jax.ShapeDtypeStruct((N, 64), jnp.float32),
    )(agg, y, dinv, b, w)


def _final_body(agg_ref, y_ref, dinv_ref, b_ref, wp1_ref, bp1_ref,
                wp2_ref, bp2_ref, wr_ref, br_ref, out_ref):
    a = jnp.concatenate([agg_ref[0], agg_ref[1]], axis=1)
    node = _leaky(dinv_ref[...] * (a + y_ref[...]) + b_ref[...], 0.2)
    h = _leaky(jnp.dot(node, wp1_ref[...], preferred_element_type=jnp.float32)
               + bp1_ref[...], 0.2)
    h = _leaky(jnp.dot(h, wp2_ref[...], preferred_element_type=jnp.float32)
               + bp2_ref[...], 0.2)
    out_ref[...] = jnp.dot(h, wr_ref[...],
                           preferred_element_type=jnp.float32) + br_ref[...]


def _final(agg, y, dinv, b, wp1, bp1, wp2, bp2, wr, br):
    N = y.shape[0]
    B = 2000
    grid = N // B
    full = lambda shp: pl.BlockSpec(shp, lambda i: (0,) * len(shp))
    return pl.pallas_call(
        _final_body,
        grid=(grid,),
        in_specs=[
            pl.BlockSpec((2, B, 32), lambda i: (0, i, 0)),
            pl.BlockSpec((B, 64), lambda i: (i, 0)),
            pl.BlockSpec((B, 1), lambda i: (i, 0)),
            full((1, 64)), full((64, 64)), full((1, 64)),
            full((64, 64)), full((1, 64)), full((64, 4)), full((1, 4)),
        ],
        out_specs=pl.BlockSpec((B, 4), lambda i: (i, 0)),
        out_shape=jax.ShapeDtypeStruct((N, 4), jnp.float32),
    )(agg, y, dinv, b, wp1, bp1, wp2, bp2, wr, br)


def _sc_deg(col2d, e2d, zd, n_pad):
    nchunk = col2d.shape[0]
    per_worker = nchunk // (NC * NS)
    iters = per_worker // SB
    rt = n_pad // NS

    def body(col_hbm, e_hbm, zd_hbm, out_hbm, cbuf, ebuf, deg_sh, sem):
        c = lax.axis_index("c")
        s = lax.axis_index("s")
        lo = s * rt
        pltpu.sync_copy(zd_hbm.at[pl.ds(lo, rt)], deg_sh.at[pl.ds(lo, rt)])
        plsc.subcore_barrier()

        wid = s * NC + c
        base0 = wid * per_worker

        def fire(base):
            pltpu.sync_copy(col_hbm.at[pl.ds(base, SB)], cbuf)
            pltpu.sync_copy(e_hbm.at[pl.ds(base, SB)], ebuf)
            for j in range(SB):
                pltpu.async_copy(ebuf.at[j], deg_sh.at[cbuf.at[j]],
                                 sem, add=True)

        def drain():
            pltpu.make_async_copy(e_hbm.at[pl.ds(0, SB)], ebuf, sem).wait()

        fire(base0)

        def it_body(it, carry):
            drain()
            fire(base0 + (it + 1) * SB)
            return carry

        lax.fori_loop(0, iters - 1, it_body, 0)
        drain()
        plsc.subcore_barrier()
        pltpu.sync_copy(deg_sh.at[pl.ds(lo, rt)],
                        out_hbm.at[pl.ds(c * n_pad + lo, rt)])

    mesh = plsc.VectorSubcoreMesh(core_axis_name="c", subcore_axis_name="s",
                                  num_cores=NC, num_subcores=NS)
    return pl.kernel(
        body,
        out_type=jax.ShapeDtypeStruct((NC * n_pad,), jnp.float32),
        mesh=mesh,
        scratch_types=[
            pltpu.VMEM((SB, CH), jnp.int32),
            pltpu.VMEM((SB, CH), jnp.float32),
            pltpu.VMEM_SHARED((n_pad,), jnp.float32),
            pltpu.SemaphoreType.DMA,
        ],
        compiler_params=pltpu.CompilerParams(use_tc_tiling_on_sc=False),
    )(col2d, e2d, zd)


def _sc_agg(yflat, row2d, col2d, e2d, za, n_pad):
    nchunk = row2d.shape[0]
    per_tile = nchunk // NS
    nsup = per_tile // SB
    assert nsup % 3 == 0 and nsup >= 9
    rt = n_pad // NS

    def body(y_hbm, row_hbm, col_hbm, e_hbm, za_hbm, out_hbm,
             rb0, rb1, rb2, cb0, cb1, cb2, eb0, eb1, eb2,
             ib0, ib1, ib2, rw0, rw1, rw2, agg_sh,
             lsem0, lsem1, lsem2, gsem0, gsem1, gsem2,
             ssem0, ssem1, ssem2):
        c = lax.axis_index("c")
        s = lax.axis_index("s")
        lo = s * rt
        pltpu.sync_copy(za_hbm.at[pl.ds(lo, rt)], agg_sh.at[pl.ds(lo, rt)])
        plsc.subcore_barrier()

        base0 = s * per_tile
        rb = (rb0, rb1, rb2)
        cb = (cb0, cb1, cb2)
        eb = (eb0, eb1, eb2)
        ib = (ib0, ib1, ib2)
        rw = (rw0, rw1, rw2)
        lsem = (lsem0, lsem1, lsem2)
        gsem = (gsem0, gsem1, gsem2)
        ssem = (ssem0, ssem1, ssem2)

        def drain_rows(sem, buf):
            pltpu.make_async_copy(y_hbm.at[pl.ds(0, SB * CH)], buf,
                                  sem).wait()

        def stage_l(sup, q, first):
            if not first:
                drain_rows(ssem[q], rw[q])
            base = base0 + sup * SB
            pltpu.async_copy(row_hbm.at[pl.ds(base, SB)], rb[q], lsem[q])
            pltpu.async_copy(col_hbm.at[pl.ds(base, SB)], cb[q], lsem[q])
            pltpu.async_copy(e_hbm.at[pl.ds(base, SB)], eb[q], lsem[q])

        def stage_g(sup, q):
            pltpu.make_async_copy(row_hbm.at[pl.ds(0, SB)], rb[q],
                                  lsem[q]).wait()
            pltpu.make_async_copy(col_hbm.at[pl.ds(0, SB)], cb[q],
                                  lsem[q]).wait()
            pltpu.make_async_copy(e_hbm.at[pl.ds(0, SB)], eb[q],
                                  lsem[q]).wait()
            for j in range(SB):
                for l in range(CH // 16):
                    v = rb[q][j, pl.ds(l * 16, 16)]
                    ib[q][j, pl.ds(l * 16, 16)] = v + v + c
                pltpu.async_copy(y_hbm.at[ib[q].at[j]],
                                 rw[q].at[pl.ds(j * CH, CH)], gsem[q])

        def stage_b(q):
            drain_rows(gsem[q], rw[q])
            for j in range(SB):

                def scale_body(g, carry2):
                    ev = eb[q][j, pl.ds(g * 16, 16)]
                    for k in range(16):
                        i = j * CH + g * 16 + k
                        w = ev[k]
                        rw[q][i, pl.ds(0, 16)] = rw[q][i, pl.ds(0, 16)] * w
                        rw[q][i, pl.ds(16, 16)] = (
                            rw[q][i, pl.ds(16, 16)] * w)
                    return carry2

                lax.fori_loop(0, CH // 16, scale_body, 0)
                pltpu.async_copy(rw[q].at[pl.ds(j * CH, CH)],
                                 agg_sh.at[cb[q].at[j]], ssem[q], add=True)

        stage_l(0, 0, True)
        stage_l(1, 1, True)
        stage_g(0, 0)
        stage_l(2, 2, True)
        stage_g(1, 1)
        stage_b(0)
        stage_l(3, 0, False)
        stage_g(2, 2)
        stage_b(1)
        stage_l(4, 1, False)
        stage_g(3, 0)
        stage_b(2)

        def it_body(t, carry):
            sup = 3 * t
            stage_l(sup + 2, 2, False)
            stage_g(sup + 1, 1)
            stage_b(0)
            stage_l(sup + 3, 0, False)
            stage_g(sup + 2, 2)
            stage_b(1)
            stage_l(sup + 4, 1, False)
            stage_g(sup + 3, 0)
            stage_b(2)
            return carry

        lax.fori_loop(1, nsup // 3 - 1, it_body, 0)
        stage_l(nsup - 1, (nsup - 1) % 3, False)
        stage_g(nsup - 2, (nsup - 2) % 3)
        stage_b((nsup - 3) % 3)
        stage_g(nsup - 1, (nsup - 1) % 3)
        stage_b((nsup - 2) % 3)
        stage_b((nsup - 1) % 3)
        drain_rows(ssem[0], rw[0])
        drain_rows(ssem[1], rw[1])
        drain_rows(ssem[2], rw[2])

        plsc.subcore_barrier()
        pltpu.sync_copy(agg_sh.at[pl.ds(lo, rt)],
                        out_hbm.at[c, pl.ds(lo, rt)])

    mesh = plsc.VectorSubcoreMesh(core_axis_name="c", subcore_axis_name="s",
                                  num_cores=NC, num_subcores=NS)
    return pl.kernel(
        body,
        out_type=jax.ShapeDtypeStruct((NC, n_pad, 32), jnp.float32),
        mesh=mesh,
        scratch_types=(
            [pltpu.VMEM((SB, CH), jnp.int32)] * 3
            + [pltpu.VMEM((SB, CH), jnp.int32)] * 3
            + [pltpu.VMEM((SB, CH), jnp.float32)] * 3
            + [pltpu.VMEM((SB, CH), jnp.int32)] * 3
            + [pltpu.VMEM((SB * CH, 32), jnp.float32)] * 3
            + [pltpu.VMEM_SHARED((n_pad, 32), jnp.float32)]
            + [pltpu.SemaphoreType.DMA] * 9
        ),
        compiler_params=pltpu.CompilerParams(use_tc_tiling_on_sc=False),
    )(yflat, row2d, col2d, e2d, za)


def _gcn_forward(x, edge_index, edge_attr, W1n, b1n, W2n, b2n, W1e, b1e,
                 W2e, b2e, gcn_W, gcn_b, Wp1, bp1, Wp2, bp2, Wr, br):
    N = x.shape[0]
    E = edge_attr.shape[0]

    rt = ((-(-N // NS)) + 127) // 128 * 128
    n_pad = rt * NS

    blk = NC * NS * SB * CH * 3
    e_pad = -(-E // blk) * blk
    nchunk = e_pad // CH
    pad = e_pad - E

    row = edge_index[0].astype(jnp.int32)
    col = edge_index[1].astype(jnp.int32)
    row2d = jnp.concatenate([row, jnp.zeros((pad,), jnp.int32)]).reshape(
        nchunk, CH)
    col2d = jnp.concatenate([col, jnp.zeros((pad,), jnp.int32)]).reshape(
        nchunk, CH)

    e = _edge_mlp(edge_attr, W1e, b1e.reshape(1, 16), W2e,
                  b2e.reshape(1, 1))
    e2d = jnp.concatenate([e.reshape(-1),
                           jnp.zeros((pad,), jnp.float32)]).reshape(nchunk, CH)

    zd = jnp.zeros((n_pad,), jnp.float32)
    za = jnp.zeros((n_pad, 32), jnp.float32)

    degp = _sc_deg(col2d, e2d, zd, n_pad).reshape(NC, n_pad, 1)

    y, dinv = _pre(x, degp, W1n, b1n.reshape(1, 64), W2n,
                   b2n.reshape(1, 64), gcn_W[0])

    out = None
    for i in range(8):
        yflat = y.reshape(2 * N, 32)
        agg = _sc_agg(yflat, row2d, col2d, e2d, za, n_pad)
        b = gcn_b[i].reshape(1, 64)
        if i < 7:
            y = _layer(agg, y, dinv, b, gcn_W[i + 1])
        else:
            out = _final(agg, y, dinv, b, Wp1, bp1.reshape(1, 64),
                         Wp2, bp2.reshape(1, 64), Wr, br.reshape(1, 4))
    return out


def kernel(x, edge_index, edge_attr, W1n, b1n, W2n, b2n, W1e, b1e, W2e,
           b2e, gcn_W, gcn_b, Wp1, bp1, Wp2, bp2, Wr, br):
    return _gcn_forward(x, edge_index, edge_attr, W1n, b1n, W2n, b2n,
                        W1e, b1e, W2e, b2e, gcn_W, gcn_b, Wp1, bp1,
                        Wp2, bp2, Wr, br)


# speedup vs baseline: 12.2682x; 1.0109x over previous
import jax
import jax.numpy as jnp
from jax import lax
from jax.experimental import pallas as pl
from jax.experimental.pallas import tpu as pltpu
from jax.experimental.pallas import tpu_sc as plsc

NC = 2
NS = 16
CH = 128
SB = 2


def _leaky(v, s):
    return jnp.where(v >= 0, v, s * v)


def _nan0(v):
    return jnp.where(jnp.isnan(v), 0.0, v)


def _edge_mlp_body(ea_ref, w1_ref, b1_ref, w2_ref, b2_ref, out_ref):
    a = _nan0(ea_ref[...])
    h = _leaky(jnp.dot(a, w1_ref[...], preferred_element_type=jnp.float32)
               + b1_ref[...], 0.2)
    o = _leaky(jnp.dot(h, w2_ref[...], preferred_element_type=jnp.float32)
               + b2_ref[...], 0.005)
    out_ref[...] = o


def _edge_mlp(ea, w1, b1, w2, b2):
    E = ea.shape[0]
    BE = 16000
    grid = E // BE
    full = lambda shp: pl.BlockSpec(shp, lambda i: (0,) * len(shp))
    return pl.pallas_call(
        _edge_mlp_body,
        grid=(grid,),
        in_specs=[
            pl.BlockSpec((BE, 4), lambda i: (i, 0)),
            full((4, 16)), full((1, 16)), full((16, 1)), full((1, 1)),
        ],
        out_specs=pl.BlockSpec((BE, 1), lambda i: (i, 0)),
        out_shape=jax.ShapeDtypeStruct((E, 1), jnp.float32),
    )(ea, w1, b1, w2, b2)


def _pre_body(x_ref, degp_ref, w1_ref, b1_ref, w2_ref, b2_ref, w0_ref,
              y_ref, dinv_ref):
    xb = _nan0(x_ref[...])
    n1 = _leaky(jnp.dot(xb, w1_ref[...], preferred_element_type=jnp.float32)
                + b1_ref[...], 0.2)
    n2 = _leaky(jnp.dot(n1, w2_ref[...], preferred_element_type=jnp.float32)
                + b2_ref[...], 0.2)
    deg = 1.0 + degp_ref[0] + degp_ref[1]
    dv = jnp.where(deg == 0.0, 0.0, lax.rsqrt(deg))
    y_ref[...] = dv * jnp.dot(n2, w0_ref[...],
                              preferred_element_type=jnp.float32)
    dinv_ref[...] = dv


def _pre(x, degp, w1, b1, w2, b2, w0):
    N = x.shape[0]
    B = 2000
    grid = N // B
    full = lambda shp: pl.BlockSpec(shp, lambda i: (0,) * len(shp))
    return pl.pallas_call(
        _pre_body,
        grid=(grid,),
        in_specs=[
            pl.BlockSpec((B, 7), lambda i: (i, 0)),
            pl.BlockSpec((2, B, 1), lambda i: (0, i, 0)),
            full((7, 64)), full((1, 64)), full((64, 64)), full((1, 64)),
            full((64, 64)),
        ],
        out_specs=[
            pl.BlockSpec((B, 64), lambda i: (i, 0)),
            pl.BlockSpec((B, 1), lambda i: (i, 0)),
        ],
        out_shape=[
            jax.ShapeDtypeStruct((N, 64), jnp.float32),
            jax.ShapeDtypeStruct((N, 1), jnp.float32),
        ],
    )(x, degp, w1, b1, w2, b2, w0)


def _layer_body(agg_ref, y_ref, dinv_ref, b_ref, w_ref, out_ref):
    a = jnp.concatenate([agg_ref[0], agg_ref[1]], axis=1)
    dv = dinv_ref[...]
    node = _leaky(dv * (a + y_ref[...]) + b_ref[...], 0.2)
    out_ref[...] = dv * jnp.dot(node, w_ref[...],
                                preferred_element_type=jnp.float32)


def _layer(agg, y, dinv, b, w):
    N = y.shape[0]
    B = 2000
    grid = N // B
    full = lambda shp: pl.BlockSpec(shp, lambda i: (0,) * len(shp))
    return pl.pallas_call(
        _layer_body,
        grid=(grid,),
        in_specs=[
            pl.BlockSpec((2, B, 32), lambda i: (0, i, 0)),
            pl.BlockSpec((B, 64), lambda i: (i, 0)),
            pl.BlockSpec((B, 1), lambda i: (i, 0)),
            full((1, 64)), full((64, 64)),
        ],
        out_specs=pl.BlockSpec((B, 64), lambda i: (i, 0)),
        out_shape=jax.ShapeDtypeStruct((N, 64), jnp.float32),
    )(agg, y, dinv, b, w)


def _final_body(agg_ref, y_ref, dinv_ref, b_ref, wp1_ref, bp1_ref,
                wp2_ref, bp2_ref, wr_ref, br_ref, out_ref):
    a = jnp.concatenate([agg_ref[0], agg_ref[1]], axis=1)
    node = _leaky(dinv_ref[...] * (a + y_ref[...]) + b_ref[...], 0.2)
    h = _leaky(jnp.dot(node, wp1_ref[...], preferred_element_type=jnp.float32)
               + bp1_ref[...], 0.2)
    h = _leaky(jnp.dot(h, wp2_ref[...], preferred_element_type=jnp.float32)
               + bp2_ref[...], 0.2)
    out_ref[...] = jnp.dot(h, wr_ref[...],
                           preferred_element_type=jnp.float32) + br_ref[...]


def _final(agg, y, dinv, b, wp1, bp1, wp2, bp2, wr, br):
    N = y.shape[0]
    B = 2000
    grid = N // B
    full = lambda shp: pl.BlockSpec(shp, lambda i: (0,) * len(shp))
    return pl.pallas_call(
        _final_body,
        grid=(grid,),
        in_specs=[
            pl.BlockSpec((2, B, 32), lambda i: (0, i, 0)),
            pl.BlockSpec((B, 64), lambda i: (i, 0)),
            pl.BlockSpec((B, 1), lambda i: (i, 0)),
            full((1, 64)), full((64, 64)), full((1, 64)),
            full((64, 64)), full((1, 64)), full((64, 4)), full((1, 4)),
        ],
        out_specs=pl.BlockSpec((B, 4), lambda i: (i, 0)),
        out_shape=jax.ShapeDtypeStruct((N, 4), jnp.float32),
    )(agg, y, dinv, b, wp1, bp1, wp2, bp2, wr, br)


def _sc_deg(col2d, e2d, zd, n_pad):
    nchunk = col2d.shape[0]
    per_worker = nchunk // (NC * NS)
    iters = per_worker // SB
    rt = n_pad // NS

    def body(col_hbm, e_hbm, zd_hbm, out_hbm, cbuf, ebuf, deg_sh, sem):
        c = lax.axis_index("c")
        s = lax.axis_index("s")
        lo = s * rt
        pltpu.sync_copy(zd_hbm.at[pl.ds(lo, rt)], deg_sh.at[pl.ds(lo, rt)])
        plsc.subcore_barrier()

        wid = s * NC + c
        base0 = wid * per_worker

        def fire(base):
            pltpu.sync_copy(col_hbm.at[pl.ds(base, SB)], cbuf)
            pltpu.sync_copy(e_hbm.at[pl.ds(base, SB)], ebuf)
            for j in range(SB):
                pltpu.async_copy(ebuf.at[j], deg_sh.at[cbuf.at[j]],
                                 sem, add=True)

        def drain():
            pltpu.make_async_copy(e_hbm.at[pl.ds(0, SB)], ebuf, sem).wait()

        fire(base0)

        def it_body(it, carry):
            drain()
            fire(base0 + (it + 1) * SB)
            return carry

        lax.fori_loop(0, iters - 1, it_body, 0)
        drain()
        plsc.subcore_barrier()
        pltpu.sync_copy(deg_sh.at[pl.ds(lo, rt)],
                        out_hbm.at[pl.ds(c * n_pad + lo, rt)])

    mesh = plsc.VectorSubcoreMesh(core_axis_name="c", subcore_axis_name="s",
                                  num_cores=NC, num_subcores=NS)
    return pl.kernel(
        body,
        out_type=jax.ShapeDtypeStruct((NC * n_pad,), jnp.float32),
        mesh=mesh,
        scratch_types=[
            pltpu.VMEM((SB, CH), jnp.int32),
            pltpu.VMEM((SB, CH), jnp.float32),
            pltpu.VMEM_SHARED((n_pad,), jnp.float32),
            pltpu.SemaphoreType.DMA,
        ],
        compiler_params=pltpu.CompilerParams(use_tc_tiling_on_sc=False),
    )(col2d, e2d, zd)


def _sc_agg(yflat, row2d, col2d, e2d, za, n_pad):
    nchunk = row2d.shape[0]
    per_tile = nchunk // NS
    nsup = per_tile // SB
    assert nsup % 3 == 0 and nsup >= 9
    rt = n_pad // NS

    def body(y_hbm, row_hbm, col_hbm, e_hbm, za_hbm, out_hbm,
             rb0, rb1, rb2, cb0, cb1, cb2, eb0, eb1, eb2,
             ib0, ib1, ib2, rw0, rw1, rw2, agg_sh,
             lsem0, lsem1, lsem2, gsem0, gsem1, gsem2,
             ssem0, ssem1, ssem2):
        c = lax.axis_index("c")
        s = lax.axis_index("s")
        lo = s * rt
        pltpu.sync_copy(za_hbm.at[pl.ds(lo, rt)], agg_sh.at[pl.ds(lo, rt)])
        plsc.subcore_barrier()

        base0 = s * per_tile
        rb = (rb0, rb1, rb2)
        cb = (cb0, cb1, cb2)
        eb = (eb0, eb1, eb2)
        ib = (ib0, ib1, ib2)
        rw = (rw0, rw1, rw2)
        lsem = (lsem0, lsem1, lsem2)
        gsem = (gsem0, gsem1, gsem2)
        ssem = (ssem0, ssem1, ssem2)

        def drain_rows(sem, buf):
            pltpu.make_async_copy(y_hbm.at[pl.ds(0, SB * CH)], buf,
                                  sem).wait()

        def stage_l(sup, q, first):
            if not first:
                drain_rows(ssem[q], rw[q])
            base = base0 + sup * SB
            pltpu.async_copy(row_hbm.at[pl.ds(base, SB)], rb[q], lsem[q])
            pltpu.async_copy(col_hbm.at[pl.ds(base, SB)], cb[q], lsem[q])
            pltpu.async_copy(e_hbm.at[pl.ds(base, SB)], eb[q], lsem[q])

        def stage_g(sup, q):
            pltpu.make_async_copy(row_hbm.at[pl.ds(0, SB)], rb[q],
                                  lsem[q]).wait()
            pltpu.make_async_copy(col_hbm.at[pl.ds(0, SB)], cb[q],
                                  lsem[q]).wait()
            pltpu.make_async_copy(e_hbm.at[pl.ds(0, SB)], eb[q],
                                  lsem[q]).wait()
            for j in range(SB):
                for l in range(CH // 16):
                    v = rb[q][j, pl.ds(l * 16, 16)]
                    ib[q][j, pl.ds(l * 16, 16)] = v + v + c
                pltpu.async_copy(y_hbm.at[ib[q].at[j]],
                                 rw[q].at[pl.ds(j * CH, CH)], gsem[q])

        def stage_b(q):
            drain_rows(gsem[q], rw[q])
            for j in range(SB):

                def scale_body(g, carry2):
                    ev = eb[q][j, pl.ds(g * 16, 16)]
                    for k in range(16):
                        i = j * CH + g * 16 + k
                        w = ev[k]
                        rw[q][i, pl.ds(0, 16)] = rw[q][i, pl.ds(0, 16)] * w
                        rw[q][i, pl.ds(16, 16)] = (
                            rw[q][i, pl.ds(16, 16)] * w)
                    return carry2

                lax.fori_loop(0, CH // 16, scale_body, 0)
                pltpu.async_copy(rw[q].at[pl.ds(j * CH, CH)],
                                 agg_sh.at[cb[q].at[j]], ssem[q], add=True)

        stage_l(0, 0, True)
        stage_l(1, 1, True)
        stage_g(0, 0)
        stage_l(2, 2, True)
        stage_g(1, 1)
        stage_b(0)
        stage_l(3, 0, False)
        stage_g(2, 2)
        stage_b(1)
        stage_l(4, 1, False)
        stage_g(3, 0)
        stage_b(2)

        def it_body(t, carry):
            sup = 3 * t
            stage_l(sup + 2, 2, False)
            stage_g(sup + 1, 1)
            stage_b(0)
            stage_l(sup + 3, 0, False)
            stage_g(sup + 2, 2)
            stage_b(1)
            stage_l(sup + 4, 1, False)
            stage_g(sup + 3, 0)
            stage_b(2)
            return carry

        lax.fori_loop(1, nsup // 3 - 1, it_body, 0)
        stage_l(nsup - 1, (nsup - 1) % 3, False)
        stage_g(nsup - 2, (nsup - 2) % 3)
        stage_b((nsup - 3) % 3)
        stage_g(nsup - 1, (nsup - 1) % 3)
        stage_b((nsup - 2) % 3)
        stage_b((nsup - 1) % 3)
        drain_rows(ssem[0], rw[0])
        drain_rows(ssem[1], rw[1])
        drain_rows(ssem[2], rw[2])

        plsc.subcore_barrier()
        pltpu.sync_copy(agg_sh.at[pl.ds(lo, rt)],
                        out_hbm.at[c, pl.ds(lo, rt)])

    mesh = plsc.VectorSubcoreMesh(core_axis_name="c", subcore_axis_name="s",
                                  num_cores=NC, num_subcores=NS)
    return pl.kernel(
        body,
        out_type=jax.ShapeDtypeStruct((NC, n_pad, 32), jnp.float32),
        mesh=mesh,
        scratch_types=(
            [pltpu.VMEM((SB, CH), jnp.int32)] * 3
            + [pltpu.VMEM((SB, CH), jnp.int32)] * 3
            + [pltpu.VMEM((SB, CH), jnp.float32)] * 3
            + [pltpu.VMEM((SB, CH), jnp.int32)] * 3
            + [pltpu.VMEM((SB * CH, 32), jnp.float32)] * 3
            + [pltpu.VMEM_SHARED((n_pad, 32), jnp.float32)]
            + [pltpu.SemaphoreType.DMA] * 9
        ),
        compiler_params=pltpu.CompilerParams(use_tc_tiling_on_sc=False),
    )(yflat, row2d, col2d, e2d, za)


def _gcn_forward(x, edge_index, edge_attr, W1n, b1n, W2n, b2n, W1e, b1e,
                 W2e, b2e, gcn_W, gcn_b, Wp1, bp1, Wp2, bp2, Wr, br):
    N = x.shape[0]
    E = edge_attr.shape[0]

    rt = ((-(-N // NS)) + 127) // 128 * 128
    n_pad = rt * NS

    blk = NC * NS * SB * CH * 3
    e_pad = -(-E // blk) * blk
    nchunk = e_pad // CH
    pad = e_pad - E

    row = edge_index[0].astype(jnp.int32)
    col = edge_index[1].astype(jnp.int32)
    row2d = jnp.concatenate([row, jnp.zeros((pad,), jnp.int32)]).reshape(
        nchunk, CH)
    col2d = jnp.concatenate([col, jnp.zeros((pad,), jnp.int32)]).reshape(
        nchunk, CH)

    e = _edge_mlp(edge_attr, W1e, b1e.reshape(1, 16), W2e,
                  b2e.reshape(1, 1))
    e2d = jnp.concatenate([e.reshape(-1),
                           jnp.zeros((pad,), jnp.float32)]).reshape(nchunk, CH)

    zd = jnp.zeros((n_pad,), jnp.float32)
    za = jnp.zeros((n_pad, 32), jnp.float32)

    degp = _sc_deg(col2d, e2d, zd, n_pad).reshape(NC, n_pad, 1)

    y, dinv = _pre(x, degp, W1n, b1n.reshape(1, 64), W2n,
                   b2n.reshape(1, 64), gcn_W[0])

    out = None
    for i in range(8):
        yflat = y.reshape(2 * N, 32)
        agg = _sc_agg(yflat, row2d, col2d, e2d, za, n_pad)
        b = gcn_b[i].reshape(1, 64)
        if i < 7:
            y = _layer(agg, y, dinv, b, gcn_W[i + 1])
        else:
            out = _final(agg, y, dinv, b, Wp1, bp1.reshape(1, 64),
                         Wp2, bp2.reshape(1, 64), Wr, br.reshape(1, 4))
    return out


def kernel(x, edge_index, edge_attr, W1n, b1n, W2n, b2n, W1e, b1e, W2e,
           b2e, gcn_W, gcn_b, Wp1, bp1, Wp2, bp2, Wr, br):
    return _gcn_forward(x, edge_index, edge_attr, W1n, b1n, W2n, b2n,
                        W1e, b1e, W2e, b2e, gcn_W, gcn_b, Wp1, bp1,
                        Wp2, bp2, Wr, br)
